# baseline (speedup 1.0000x reference)
import functools

import jax
import jax.numpy as jnp
from jax import lax
from jax.experimental import pallas as pl
from jax.experimental.pallas import tpu as pltpu
from jax.experimental.pallas import tpu_sc as plsc

N = 10000
D = 128
NC = 2
NS = 16
NW = NC * NS
K = 128
CH = 81
NB = 3

_mesh = plsc.VectorSubcoreMesh(
    core_axis_name="c", subcore_axis_name="s", num_cores=NC, num_subcores=NS
)


@functools.partial(
    pl.kernel,
    out_type=jax.ShapeDtypeStruct((NC * N, D), jnp.float32),
    mesh=_mesh,
    scratch_types=(
        [pltpu.VMEM((K,), jnp.int32) for _ in range(NB)]
        + [pltpu.VMEM((K,), jnp.int32) for _ in range(NB)]
        + [pltpu.VMEM((K,), jnp.float32) for _ in range(NB)]
        + [pltpu.VMEM((K, D), jnp.float32) for _ in range(NB)]
        + [pltpu.VMEM_SHARED((N, D), jnp.float32)]
        + [pltpu.SemaphoreType.DMA for _ in range(2 * NB)]
    ),
)
def _sc_spmm(row_hbm, col_hbm, val_hbm, x_hbm, out_hbm, *scratch):
    cbufs = scratch[0:NB]
    rbufs = scratch[NB:2 * NB]
    vbufs = scratch[2 * NB:3 * NB]
    bufs = scratch[3 * NB:4 * NB]
    acc = scratch[4 * NB]
    isems = scratch[4 * NB + 1:4 * NB + 1 + NB]
    gsems = scratch[4 * NB + 1 + NB:4 * NB + 1 + 2 * NB]
    buf0 = bufs[0]

    c = lax.axis_index("c")
    s = lax.axis_index("s")
    w = s * NC + c
    ebase = w * CH * K

    def _zrow(r, carry):
        for j in range(D // 16):
            buf0[r, pl.ds(j * 16, 16)] = jnp.zeros((16,), jnp.float32)
        return carry

    lax.fori_loop(0, K, _zrow, 0)
    for z in range(5):
        idx = s + z * NS

        @pl.when(idx < N // K)
        def _():
            pltpu.sync_copy(buf0, acc.at[pl.ds(idx * K, K)])

    @pl.when(s == 0)
    def _():
        pltpu.sync_copy(buf0.at[pl.ds(0, N % K)],
                        acc.at[pl.ds((N // K) * K, N % K)])

    plsc.subcore_barrier()

    def _idx_dma(k, b):
        off = ebase + k * K
        pltpu.async_copy(col_hbm.at[pl.ds(off, K)], cbufs[b], isems[b])
        pltpu.async_copy(row_hbm.at[pl.ds(off, K)], rbufs[b], isems[b])
        pltpu.async_copy(val_hbm.at[pl.ds(off, K)], vbufs[b], isems[b])

    def _idx_wait(b):
        z = pl.ds(0, K)
        pltpu.make_async_copy(col_hbm.at[z], cbufs[b], isems[b]).wait()
        pltpu.make_async_copy(row_hbm.at[z], rbufs[b], isems[b]).wait()
        pltpu.make_async_copy(val_hbm.at[z], vbufs[b], isems[b]).wait()

    def _gather(b):
        pltpu.async_copy(x_hbm.at[cbufs[b]], bufs[b], gsems[b])

    def _gwait(b):
        pltpu.make_async_copy(x_hbm.at[cbufs[b]], bufs[b], gsems[b]).wait()

    def _process(b):
        def _edge16(g, inner):
            vv = vbufs[b][pl.ds(g * 16, 16)]
            for i in range(16):
                e = g * 16 + i
                v = vv[i]
                for j in range(D // 16):
                    sl = pl.ds(j * 16, 16)
                    bufs[b][e, sl] = bufs[b][e, sl] * v
            return inner

        lax.fori_loop(0, K // 16, _edge16, 0)
        pltpu.sync_copy(bufs[b], acc.at[rbufs[b]], add=True)

    for b in range(NB):
        _idx_dma(b, b)
    for b in range(NB - 1):
        _idx_wait(b)
        _gather(b)

    def _round(p, carry):
        for b in range(NB):
            k = NB * p + b
            _gwait(b)
            _idx_wait((b + NB - 1) % NB)
            _gather((b + NB - 1) % NB)
            _process(b)
            _idx_dma(jnp.minimum(k + NB, CH - 1), b)
        return carry

    lax.fori_loop(0, CH // NB, _round, 0)
    _gwait(CH % NB)
    _gwait((CH + 1) % NB)
    _idx_wait((CH - 1) % NB)

    plsc.subcore_barrier()

    for z in range(5):
        idx = s + z * NS

        @pl.when(idx < N // K)
        def _():
            pltpu.sync_copy(acc.at[pl.ds(idx * K, K)],
                            out_hbm.at[pl.ds(c * N + idx * K, K)])

    @pl.when(s == 0)
    def _():
        pltpu.sync_copy(acc.at[pl.ds((N // K) * K, N % K)],
                        out_hbm.at[pl.ds(c * N + (N // K) * K, N % K)])


def _add_body(a_ref, b_ref, o_ref):
    o_ref[...] = a_ref[...] + b_ref[...]


def _combine_partials(partial):
    br = 400
    grid = N // br
    return pl.pallas_call(
        _add_body,
        out_shape=jax.ShapeDtypeStruct((N, D), jnp.float32),
        grid=(grid,),
        in_specs=[
            pl.BlockSpec((br, D), lambda i: (i, 0)),
            pl.BlockSpec((br, D), lambda i: (i + grid, 0)),
        ],
        out_specs=pl.BlockSpec((br, D), lambda i: (i, 0)),
    )(partial, partial)


def kernel(adj_indices, adj_values, x):
    row = adj_indices[0]
    col = adj_indices[1]
    e = adj_values.shape[0]
    ep = NW * CH * K
    pad = ep - e
    row_p = jnp.concatenate([row, jnp.zeros((pad,), jnp.int32)])
    col_p = jnp.concatenate([col, jnp.zeros((pad,), jnp.int32)])
    val_p = jnp.concatenate([adj_values, jnp.zeros((pad,), jnp.float32)])
    partial = _sc_spmm(row_p, col_p, val_p, x)
    return _combine_partials(partial)


# speedup vs baseline: 1.2754x; 1.2754x over previous
import functools

import jax
import jax.numpy as jnp
from jax import lax
from jax.experimental import pallas as pl
from jax.experimental.pallas import tpu as pltpu
from jax.experimental.pallas import tpu_sc as plsc

N = 10000
D = 128
NC = 2
NS = 16
NW = NC * NS
K = 128
CH = 80
NB = 2

_mesh = plsc.VectorSubcoreMesh(
    core_axis_name="c", subcore_axis_name="s", num_cores=NC, num_subcores=NS
)


@functools.partial(
    pl.kernel,
    out_type=jax.ShapeDtypeStruct((NC * N, D), jnp.float32),
    mesh=_mesh,
    scratch_types=(
        [pltpu.VMEM((K,), jnp.int32) for _ in range(NB)]
        + [pltpu.VMEM((K,), jnp.int32) for _ in range(NB)]
        + [pltpu.VMEM((K,), jnp.float32) for _ in range(NB)]
        + [pltpu.VMEM((K, D), jnp.float32) for _ in range(NB)]
        + [pltpu.VMEM_SHARED((N, D), jnp.float32)]
        + [pltpu.SemaphoreType.DMA for _ in range(2 * NB)]
    ),
)
def _sc_spmm(row_hbm, col_hbm, val_hbm, x_hbm, out_hbm, *scratch):
    cbufs = scratch[0:NB]
    rbufs = scratch[NB:2 * NB]
    vbufs = scratch[2 * NB:3 * NB]
    bufs = scratch[3 * NB:4 * NB]
    acc = scratch[4 * NB]
    isems = scratch[4 * NB + 1:4 * NB + 1 + NB]
    gsems = scratch[4 * NB + 1 + NB:4 * NB + 1 + 2 * NB]
    buf0 = bufs[0]

    c = lax.axis_index("c")
    s = lax.axis_index("s")
    w = s * NC + c
    ebase = w * CH * K

    def _zrow(r, carry):
        for j in range(D // 16):
            buf0[r, pl.ds(j * 16, 16)] = jnp.zeros((16,), jnp.float32)
        return carry

    lax.fori_loop(0, K, _zrow, 0)
    for z in range(5):
        idx = s + z * NS

        @pl.when(idx < N // K)
        def _():
            pltpu.sync_copy(buf0, acc.at[pl.ds(idx * K, K)])

    @pl.when(s == 0)
    def _():
        pltpu.sync_copy(buf0.at[pl.ds(0, N % K)],
                        acc.at[pl.ds((N // K) * K, N % K)])

    plsc.subcore_barrier()

    def _idx_dma(k, b):
        off = ebase + k * K
        pltpu.async_copy(col_hbm.at[pl.ds(off, K)], cbufs[b], isems[b])
        pltpu.async_copy(row_hbm.at[pl.ds(off, K)], rbufs[b], isems[b])
        pltpu.async_copy(val_hbm.at[pl.ds(off, K)], vbufs[b], isems[b])

    def _idx_wait(b):
        z = pl.ds(0, K)
        pltpu.make_async_copy(col_hbm.at[z], cbufs[b], isems[b]).wait()
        pltpu.make_async_copy(row_hbm.at[z], rbufs[b], isems[b]).wait()
        pltpu.make_async_copy(val_hbm.at[z], vbufs[b], isems[b]).wait()

    def _gather(b):
        pltpu.async_copy(x_hbm.at[cbufs[b]], bufs[b], gsems[b])

    def _gwait(b):
        pltpu.make_async_copy(x_hbm.at[cbufs[b]], bufs[b], gsems[b]).wait()

    def _process(b):
        def _edge16(g, inner):
            vv = vbufs[b][pl.ds(g * 16, 16)]
            for i in range(16):
                e = g * 16 + i
                v = vv[i]
                for j in range(D // 16):
                    sl = pl.ds(j * 16, 16)
                    bufs[b][e, sl] = bufs[b][e, sl] * v
            return inner

        lax.fori_loop(0, K // 16, _edge16, 0)
        pltpu.sync_copy(bufs[b], acc.at[rbufs[b]], add=True)

    for b in range(NB):
        _idx_dma(b, b)
    for b in range(NB - 1):
        _idx_wait(b)
        _gather(b)

    def _round(p, carry):
        for b in range(NB):
            k = NB * p + b
            _gwait(b)
            _idx_wait((b + NB - 1) % NB)
            _gather((b + NB - 1) % NB)
            _process(b)
            _idx_dma(jnp.minimum(k + NB, CH - 1), b)
        return carry

    lax.fori_loop(0, CH // NB, _round, 0)
    for i in range(NB - 1):
        _gwait((CH + i) % NB)
    _idx_wait((CH - 1) % NB)

    plsc.subcore_barrier()

    for z in range(5):
        idx = s + z * NS

        @pl.when(idx < N // K)
        def _():
            pltpu.sync_copy(acc.at[pl.ds(idx * K, K)],
                            out_hbm.at[pl.ds(c * N + idx * K, K)])

    @pl.when(s == 0)
    def _():
        pltpu.sync_copy(acc.at[pl.ds((N // K) * K, N % K)],
                        out_hbm.at[pl.ds(c * N + (N // K) * K, N % K)])


def _add_body(a_ref, b_ref, o_ref):
    o_ref[...] = a_ref[...] + b_ref[...]


def _combine_partials(partial):
    br = 400
    grid = N // br
    return pl.pallas_call(
        _add_body,
        out_shape=jax.ShapeDtypeStruct((N, D), jnp.float32),
        grid=(grid,),
        in_specs=[
            pl.BlockSpec((br, D), lambda i: (i, 0)),
            pl.BlockSpec((br, D), lambda i: (i + grid, 0)),
        ],
        out_specs=pl.BlockSpec((br, D), lambda i: (i, 0)),
    )(partial, partial)


def kernel(adj_indices, adj_values, x):
    row = adj_indices[0]
    col = adj_indices[1]
    e = adj_values.shape[0]
    ep = NW * CH * K
    pad = ep - e
    row_p = jnp.concatenate([row, jnp.zeros((pad,), jnp.int32)])
    col_p = jnp.concatenate([col, jnp.zeros((pad,), jnp.int32)])
    val_p = jnp.concatenate([adj_values, jnp.zeros((pad,), jnp.float32)])
    partial = _sc_spmm(row_p, col_p, val_p, x)
    return _combine_partials(partial)
